# initial kernel scaffold (unmeasured)
import jax
import jax.numpy as jnp
from jax import lax
from jax.experimental import pallas as pl
from jax.experimental.pallas import tpu as pltpu

N_DEV = 8
B = 4
Sq = 256
Hq = 8
Hkv = 2
Dh = 128
Dm = 1024
C = 1024
G = Hq // Hkv
R = G * Sq
SCALE = 0.08838834764831843

_MESH = pl.DeviceIdType.MESH


def kernel(x, Wq, Wo, K_ext, V_ext):
    def body(x_ref, wq_ref, wo_ref, k_ref, v_ref, out_ref,
             accs_ref, mls_ref, acc_run_ref, attn_ref,
             acc_send_sems, acc_recv_sems, ml_send_sems, ml_recv_sems,
             credit_sem):
        my = lax.axis_index("i")
        left = lax.rem(my + N_DEV - 1, N_DEV)
        right = lax.rem(my + 1, N_DEV)

        barrier = pltpu.get_barrier_semaphore()
        for nbr in (left, right):
            pl.semaphore_signal(barrier, inc=1, device_id=(nbr,),
                                device_id_type=_MESH)
        pl.semaphore_wait(barrier, 2)

        xq = jnp.dot(x_ref[:].reshape(B * Sq, Dm), wq_ref[:],
                     preferred_element_type=jnp.float32) * SCALE

        m_run = [[None] * Hkv for _ in range(B)]
        l_run = [[None] * Hkv for _ in range(B)]
        for b in range(B):
            for g in range(Hkv):
                q_bg = jnp.concatenate(
                    [xq[b * Sq:(b + 1) * Sq,
                        (g * G + qi) * Dh:(g * G + qi + 1) * Dh]
                     for qi in range(G)], axis=0)
                kg = k_ref[b, :, g, :]
                s = lax.dot_general(
                    q_bg, kg, (((1,), (1,)), ((), ())),
                    preferred_element_type=jnp.float32)
                m_b = jnp.max(s, axis=1, keepdims=True)
                p = jnp.exp(s - m_b)
                l_b = jnp.sum(p, axis=1, keepdims=True)
                vg = v_ref[b, :, g, :]
                acc = jnp.dot(p, vg,
                              preferred_element_type=jnp.float32)
                acc_run_ref[b, g, :, :] = acc
                accs_ref[0, b, g, :, :] = acc
                mls_ref[0, b, g, :, :] = jnp.concatenate([m_b, l_b], axis=1)
                m_run[b][g] = m_b
                l_run[b][g] = l_b

        for h in range(N_DEV - 1):
            ss = h % 2
            rs = (h + 1) % 2
            if h >= 2:
                pl.semaphore_wait(credit_sem, 1)
            acc_rdma = pltpu.make_async_remote_copy(
                src_ref=accs_ref.at[ss], dst_ref=accs_ref.at[rs],
                send_sem=acc_send_sems.at[ss], recv_sem=acc_recv_sems.at[rs],
                device_id=(right,), device_id_type=_MESH)
            ml_rdma = pltpu.make_async_remote_copy(
                src_ref=mls_ref.at[ss], dst_ref=mls_ref.at[rs],
                send_sem=ml_send_sems.at[ss], recv_sem=ml_recv_sems.at[rs],
                device_id=(right,), device_id_type=_MESH)
            acc_rdma.start()
            ml_rdma.start()
            acc_rdma.wait()
            ml_rdma.wait()
            if h <= N_DEV - 4:
                pl.semaphore_signal(credit_sem, inc=1, device_id=(left,),
                                    device_id_type=_MESH)
            for b in range(B):
                for g in range(Hkv):
                    ml_in = mls_ref[rs, b, g, :, :]
                    m_in = ml_in[:, 0:1]
                    l_in = ml_in[:, 1:2]
                    m_old = m_run[b][g]
                    m_new = jnp.maximum(m_old, m_in)
                    a_old = jnp.exp(m_old - m_new)
                    a_in = jnp.exp(m_in - m_new)
                    acc_run_ref[b, g, :, :] = (
                        acc_run_ref[b, g, :, :] * a_old
                        + accs_ref[rs, b, g, :, :] * a_in)
                    l_run[b][g] = l_run[b][g] * a_old + l_in * a_in
                    m_run[b][g] = m_new

        for b in range(B):
            for g in range(Hkv):
                o_bg = acc_run_ref[b, g, :, :] / l_run[b][g]
                for qi in range(G):
                    attn_ref[b * Sq:(b + 1) * Sq,
                             (g * G + qi) * Dh:(g * G + qi + 1) * Dh] = (
                        o_bg[qi * Sq:(qi + 1) * Sq, :])

        res = jnp.dot(attn_ref[:], wo_ref[:],
                      preferred_element_type=jnp.float32)
        out_ref[:] = res.reshape(B, Sq, Dm)

    return pl.pallas_call(
        body,
        out_shape=jax.ShapeDtypeStruct((B, Sq, Dm), jnp.float32),
        in_specs=[pl.BlockSpec(memory_space=pltpu.VMEM)] * 5,
        out_specs=pl.BlockSpec(memory_space=pltpu.VMEM),
        scratch_shapes=[
            pltpu.VMEM((2, B, Hkv, R, Dh), jnp.float32),
            pltpu.VMEM((2, B, Hkv, R, 2), jnp.float32),
            pltpu.VMEM((B, Hkv, R, Dh), jnp.float32),
            pltpu.VMEM((B * Sq, Dm), jnp.float32),
            pltpu.SemaphoreType.DMA((2,)),
            pltpu.SemaphoreType.DMA((2,)),
            pltpu.SemaphoreType.DMA((2,)),
            pltpu.SemaphoreType.DMA((2,)),
            pltpu.SemaphoreType.REGULAR,
        ],
        compiler_params=pltpu.CompilerParams(collective_id=0),
    )(x, Wq, Wo, K_ext, V_ext)


# baseline (device time: 378928 ns/iter reference)
import jax
import jax.numpy as jnp
from jax import lax
from jax.experimental import pallas as pl
from jax.experimental.pallas import tpu as pltpu

N_DEV = 8
B = 4
Sq = 256
Hq = 8
Hkv = 2
Dh = 128
Dm = 1024
C = 1024
G = Hq // Hkv
R = G * Sq
SCALE = 0.08838834764831843

_MESH = pl.DeviceIdType.MESH


def kernel(x, Wq, Wo, K_ext, V_ext):
    def body(x_ref, wq_ref, wo_ref, k_ref, v_ref, out_ref,
             accs_ref, mls_ref, acc_run_ref,
             acc_send_sems, acc_recv_sems, ml_send_sems, ml_recv_sems,
             credit_sem):
        my = lax.axis_index("i")
        left = lax.rem(my + N_DEV - 1, N_DEV)
        right = lax.rem(my + 1, N_DEV)

        barrier = pltpu.get_barrier_semaphore()
        for nbr in (left, right):
            pl.semaphore_signal(barrier, inc=1, device_id=(nbr,),
                                device_id_type=_MESH)
        pl.semaphore_wait(barrier, 2)

        xqT = lax.dot_general(
            wq_ref[:], x_ref[:].reshape(B * Sq, Dm),
            (((0,), (1,)), ((), ())),
            preferred_element_type=jnp.float32) * SCALE

        m_run = [[None] * Hkv for _ in range(B)]
        l_run = [[None] * Hkv for _ in range(B)]
        for b in range(B):
            for g in range(Hkv):
                qT = jnp.concatenate(
                    [xqT[(g * G + qi) * Dh:(g * G + qi + 1) * Dh,
                         b * Sq:(b + 1) * Sq]
                     for qi in range(G)], axis=1)
                kg = k_ref[b, :, g, :]
                sT = lax.dot_general(
                    kg, qT, (((1,), (0,)), ((), ())),
                    preferred_element_type=jnp.float32)
                m_b = jnp.max(sT, axis=0, keepdims=True)
                pT = jnp.exp(sT - m_b)
                l_b = jnp.sum(pT, axis=0, keepdims=True)
                vg = v_ref[b, :, g, :]
                accT = lax.dot_general(
                    vg, pT, (((0,), (0,)), ((), ())),
                    preferred_element_type=jnp.float32)
                acc_run_ref[b, g, :, :] = accT
                accs_ref[0, b, g, :, :] = accT
                mls_ref[0, b, g, :, :] = jnp.concatenate([m_b, l_b], axis=0)
                m_run[b][g] = m_b
                l_run[b][g] = l_b

        for h in range(N_DEV - 1):
            ss = h % 2
            rs = (h + 1) % 2
            acc_rdma = pltpu.make_async_remote_copy(
                src_ref=accs_ref.at[ss], dst_ref=accs_ref.at[rs],
                send_sem=acc_send_sems.at[ss], recv_sem=acc_recv_sems.at[rs],
                device_id=(right,), device_id_type=_MESH)
            ml_rdma = pltpu.make_async_remote_copy(
                src_ref=mls_ref.at[ss], dst_ref=mls_ref.at[rs],
                send_sem=ml_send_sems.at[ss], recv_sem=ml_recv_sems.at[rs],
                device_id=(right,), device_id_type=_MESH)
            acc_rdma.start()
            ml_rdma.start()
            acc_rdma.wait()
            ml_rdma.wait()
            for b in range(B):
                for g in range(Hkv):
                    m_in = mls_ref[rs, b, g, 0:1, :]
                    l_in = mls_ref[rs, b, g, 1:2, :]
                    m_old = m_run[b][g]
                    m_new = jnp.maximum(m_old, m_in)
                    a_old = jnp.exp(m_old - m_new)
                    a_in = jnp.exp(m_in - m_new)
                    acc_run_ref[b, g, :, :] = (
                        acc_run_ref[b, g, :, :] * a_old
                        + accs_ref[rs, b, g, :, :] * a_in)
                    l_run[b][g] = l_run[b][g] * a_old + l_in * a_in
                    m_run[b][g] = m_new

        for b in range(B):
            total = None
            for g in range(Hkv):
                inv_l = 1.0 / l_run[b][g]
                for qi in range(G):
                    h_idx = g * G + qi
                    oT = (acc_run_ref[b, g, :, qi * Sq:(qi + 1) * Sq]
                          * inv_l[:, qi * Sq:(qi + 1) * Sq])
                    contrib = lax.dot_general(
                        oT, wo_ref[h_idx * Dh:(h_idx + 1) * Dh, :],
                        (((0,), (0,)), ((), ())),
                        preferred_element_type=jnp.float32)
                    total = contrib if total is None else total + contrib
            out_ref[b, :, :] = total

    return pl.pallas_call(
        body,
        out_shape=jax.ShapeDtypeStruct((B, Sq, Dm), jnp.float32),
        in_specs=[pl.BlockSpec(memory_space=pltpu.VMEM)] * 5,
        out_specs=pl.BlockSpec(memory_space=pltpu.VMEM),
        scratch_shapes=[
            pltpu.VMEM((2, B, Hkv, Dh, R), jnp.float32),
            pltpu.VMEM((2, B, Hkv, 2, R), jnp.float32),
            pltpu.VMEM((B, Hkv, Dh, R), jnp.float32),
            pltpu.SemaphoreType.DMA((2,)),
            pltpu.SemaphoreType.DMA((2,)),
            pltpu.SemaphoreType.DMA((2,)),
            pltpu.SemaphoreType.DMA((2,)),
            pltpu.SemaphoreType.REGULAR,
        ],
        compiler_params=pltpu.CompilerParams(
            collective_id=0, vmem_limit_bytes=100 * 1024 * 1024),
    )(x, Wq, Wo, K_ext, V_ext)


# device time: 224319 ns/iter; 1.6892x vs baseline; 1.6892x over previous
import jax
import jax.numpy as jnp
from jax import lax
from jax.experimental import pallas as pl
from jax.experimental.pallas import tpu as pltpu

N_DEV = 8
B = 4
Sq = 256
Hq = 8
Hkv = 2
Dh = 128
Dm = 1024
C = 1024
G = Hq // Hkv
R = G * Sq
SCALE = 0.08838834764831843
FWD_HOPS = N_DEV // 2
BWD_HOPS = N_DEV - 1 - FWD_HOPS

_MESH = pl.DeviceIdType.MESH


def _gray(v):
    return jnp.where(v < 4, v, 11 - v)


def kernel(x, Wq, Wo, K_ext, V_ext):
    def body(x_ref, wq_ref, wo_ref, k_ref, v_ref, out_ref,
             rbuf, lbuf, rl, ll, acc_run_ref,
             r_send, r_recv, l_send, l_recv,
             rl_send, rl_recv, ll_send, ll_recv):
        my = lax.axis_index("i")
        vi = _gray(my)
        right = _gray(lax.rem(vi + 1, N_DEV))
        left = _gray(lax.rem(vi + N_DEV - 1, N_DEV))

        barrier = pltpu.get_barrier_semaphore()
        for nbr in (left, right):
            pl.semaphore_signal(barrier, inc=1, device_id=(nbr,),
                                device_id_type=_MESH)
        pl.semaphore_wait(barrier, 2)

        xqT = lax.dot_general(
            wq_ref[:], x_ref[:].reshape(B * Sq, Dm),
            (((0,), (1,)), ((), ())),
            preferred_element_type=jnp.float32) * SCALE

        l_run = [[None] * Hkv for _ in range(B)]
        for b in range(B):
            for g in range(Hkv):
                qT = jnp.concatenate(
                    [xqT[(g * G + qi) * Dh:(g * G + qi + 1) * Dh,
                         b * Sq:(b + 1) * Sq]
                     for qi in range(G)], axis=1)
                kg = k_ref[b, :, g, :]
                sT = lax.dot_general(
                    kg, qT, (((1,), (0,)), ((), ())),
                    preferred_element_type=jnp.float32)
                pT = jnp.exp(sT)
                l_b = jnp.sum(pT, axis=0, keepdims=True)
                vg = v_ref[b, :, g, :]
                accT = lax.dot_general(
                    vg, pT, (((0,), (0,)), ((), ())),
                    preferred_element_type=jnp.float32)
                acc_run_ref[b, g, :, :] = accT
                rbuf[0, b, g, :, :] = accT
                lbuf[0, b, g, :, :] = accT
                rl[0, b, g, :, :] = l_b
                ll[0, b, g, :, :] = l_b
                l_run[b][g] = l_b

        def make(buf, sems_s, sems_r, ss, rs, dst):
            return pltpu.make_async_remote_copy(
                src_ref=buf.at[ss], dst_ref=buf.at[rs],
                send_sem=sems_s.at[ss], recv_sem=sems_r.at[rs],
                device_id=(dst,), device_id_type=_MESH)

        def start_hop(h):
            ss, rs = h % 2, (h + 1) % 2
            rdmas = []
            if h < FWD_HOPS:
                rdmas.append(make(rbuf, r_send, r_recv, ss, rs, right))
                rdmas.append(make(rl, rl_send, rl_recv, ss, rs, right))
            if h < BWD_HOPS:
                rdmas.append(make(lbuf, l_send, l_recv, ss, rs, left))
                rdmas.append(make(ll, ll_send, ll_recv, ss, rs, left))
            for r in rdmas:
                r.start()
            return rdmas

        inflight = start_hop(0)
        for h in range(FWD_HOPS):
            rs = (h + 1) % 2
            for r in inflight:
                r.wait()
            inflight = start_hop(h + 1) if h + 1 < FWD_HOPS else []
            for b in range(B):
                for g in range(Hkv):
                    inc = rbuf[rs, b, g, :, :]
                    l_inc = rl[rs, b, g, :, :]
                    if h < BWD_HOPS:
                        inc = inc + lbuf[rs, b, g, :, :]
                        l_inc = l_inc + ll[rs, b, g, :, :]
                    acc_run_ref[b, g, :, :] = acc_run_ref[b, g, :, :] + inc
                    l_run[b][g] = l_run[b][g] + l_inc

        for b in range(B):
            total = None
            for g in range(Hkv):
                inv_l = 1.0 / l_run[b][g]
                for qi in range(G):
                    h_idx = g * G + qi
                    oT = (acc_run_ref[b, g, :, qi * Sq:(qi + 1) * Sq]
                          * inv_l[:, qi * Sq:(qi + 1) * Sq])
                    contrib = lax.dot_general(
                        oT, wo_ref[h_idx * Dh:(h_idx + 1) * Dh, :],
                        (((0,), (0,)), ((), ())),
                        preferred_element_type=jnp.float32)
                    total = contrib if total is None else total + contrib
            out_ref[b, :, :] = total

    return pl.pallas_call(
        body,
        out_shape=jax.ShapeDtypeStruct((B, Sq, Dm), jnp.float32),
        in_specs=[pl.BlockSpec(memory_space=pltpu.VMEM)] * 5,
        out_specs=pl.BlockSpec(memory_space=pltpu.VMEM),
        scratch_shapes=[
            pltpu.VMEM((2, B, Hkv, Dh, R), jnp.float32),
            pltpu.VMEM((2, B, Hkv, Dh, R), jnp.float32),
            pltpu.VMEM((2, B, Hkv, 1, R), jnp.float32),
            pltpu.VMEM((2, B, Hkv, 1, R), jnp.float32),
            pltpu.VMEM((B, Hkv, Dh, R), jnp.float32),
            pltpu.SemaphoreType.DMA((2,)),
            pltpu.SemaphoreType.DMA((2,)),
            pltpu.SemaphoreType.DMA((2,)),
            pltpu.SemaphoreType.DMA((2,)),
            pltpu.SemaphoreType.DMA((2,)),
            pltpu.SemaphoreType.DMA((2,)),
            pltpu.SemaphoreType.DMA((2,)),
            pltpu.SemaphoreType.DMA((2,)),
        ],
        compiler_params=pltpu.CompilerParams(
            collective_id=0, vmem_limit_bytes=100 * 1024 * 1024),
    )(x, Wq, Wo, K_ext, V_ext)


# device time: 134257 ns/iter; 2.8224x vs baseline; 1.6708x over previous
import jax
import jax.numpy as jnp
from jax import lax
from jax.experimental import pallas as pl
from jax.experimental.pallas import tpu as pltpu

N_DEV = 8
B = 4
Sq = 256
Hq = 8
Hkv = 2
Dh = 128
Dm = 1024
C = 1024
G = Hq // Hkv
R = G * Sq
SCALE = 0.08838834764831843
FWD_HOPS = N_DEV // 2
BWD_HOPS = N_DEV - 1 - FWD_HOPS

_MESH = pl.DeviceIdType.MESH


def _gray(v):
    return jnp.where(v < 4, v, 11 - v)


def kernel(x, Wq, Wo, K_ext, V_ext):
    def body(x_ref, wq_ref, wo_ref, k_ref, v_ref, out_ref,
             rbuf, lbuf, rl, ll, acc_run_ref,
             r_send, r_recv, l_send, l_recv,
             rl_send, rl_recv, ll_send, ll_recv):
        my = lax.axis_index("i")
        vi = _gray(my)
        right = _gray(lax.rem(vi + 1, N_DEV))
        left = _gray(lax.rem(vi + N_DEV - 1, N_DEV))

        barrier = pltpu.get_barrier_semaphore()
        for nbr in (left, right):
            pl.semaphore_signal(barrier, inc=1, device_id=(nbr,),
                                device_id_type=_MESH)
        pl.semaphore_wait(barrier, 2)

        xqT = lax.dot_general(
            wq_ref[:], x_ref[:].reshape(B * Sq, Dm),
            (((0,), (1,)), ((), ())),
            preferred_element_type=jnp.float32) * SCALE

        l_run = [[None] * Hkv for _ in range(B)]
        for b in range(B):
            for g in range(Hkv):
                qT = jnp.concatenate(
                    [xqT[(g * G + qi) * Dh:(g * G + qi + 1) * Dh,
                         b * Sq:(b + 1) * Sq]
                     for qi in range(G)], axis=1)
                kg = k_ref[b, :, g, :]
                sT = lax.dot_general(
                    kg, qT, (((1,), (0,)), ((), ())),
                    preferred_element_type=jnp.float32)
                pT = jnp.exp(sT)
                l_b = jnp.sum(pT, axis=0, keepdims=True)
                vg = v_ref[b, :, g, :]
                accT = lax.dot_general(
                    vg, pT, (((0,), (0,)), ((), ())),
                    preferred_element_type=jnp.float32)
                acc_run_ref[b, g, :, :] = accT
                acc16 = accT.astype(jnp.bfloat16)
                l16 = l_b.astype(jnp.bfloat16)
                rbuf[0, b, g, :, :] = acc16
                lbuf[0, b, g, :, :] = acc16
                rl[0, b, g, :, :] = l16
                ll[0, b, g, :, :] = l16
                l_run[b][g] = l_b

        def make(buf, sems_s, sems_r, ss, rs, dst):
            return pltpu.make_async_remote_copy(
                src_ref=buf.at[ss], dst_ref=buf.at[rs],
                send_sem=sems_s.at[ss], recv_sem=sems_r.at[rs],
                device_id=(dst,), device_id_type=_MESH)

        def start_hop(h):
            ss, rs = h % 2, (h + 1) % 2
            rdmas = []
            if h < FWD_HOPS:
                rdmas.append(make(rbuf, r_send, r_recv, ss, rs, right))
                rdmas.append(make(rl, rl_send, rl_recv, ss, rs, right))
            if h < BWD_HOPS:
                rdmas.append(make(lbuf, l_send, l_recv, ss, rs, left))
                rdmas.append(make(ll, ll_send, ll_recv, ss, rs, left))
            for r in rdmas:
                r.start()
            return rdmas

        inflight = start_hop(0)
        for h in range(FWD_HOPS):
            rs = (h + 1) % 2
            for r in inflight:
                r.wait()
            inflight = start_hop(h + 1) if h + 1 < FWD_HOPS else []
            for b in range(B):
                for g in range(Hkv):
                    inc = rbuf[rs, b, g, :, :].astype(jnp.float32)
                    l_inc = rl[rs, b, g, :, :].astype(jnp.float32)
                    if h < BWD_HOPS:
                        inc = inc + lbuf[rs, b, g, :, :].astype(jnp.float32)
                        l_inc = l_inc + ll[rs, b, g, :, :].astype(jnp.float32)
                    acc_run_ref[b, g, :, :] = acc_run_ref[b, g, :, :] + inc
                    l_run[b][g] = l_run[b][g] + l_inc

        for b in range(B):
            total = None
            for g in range(Hkv):
                inv_l = 1.0 / l_run[b][g]
                for qi in range(G):
                    h_idx = g * G + qi
                    oT = (acc_run_ref[b, g, :, qi * Sq:(qi + 1) * Sq]
                          * inv_l[:, qi * Sq:(qi + 1) * Sq])
                    contrib = lax.dot_general(
                        oT, wo_ref[h_idx * Dh:(h_idx + 1) * Dh, :],
                        (((0,), (0,)), ((), ())),
                        preferred_element_type=jnp.float32)
                    total = contrib if total is None else total + contrib
            out_ref[b, :, :] = total

    return pl.pallas_call(
        body,
        out_shape=jax.ShapeDtypeStruct((B, Sq, Dm), jnp.float32),
        in_specs=[pl.BlockSpec(memory_space=pltpu.VMEM)] * 5,
        out_specs=pl.BlockSpec(memory_space=pltpu.VMEM),
        scratch_shapes=[
            pltpu.VMEM((2, B, Hkv, Dh, R), jnp.bfloat16),
            pltpu.VMEM((2, B, Hkv, Dh, R), jnp.bfloat16),
            pltpu.VMEM((2, B, Hkv, 1, R), jnp.bfloat16),
            pltpu.VMEM((2, B, Hkv, 1, R), jnp.bfloat16),
            pltpu.VMEM((B, Hkv, Dh, R), jnp.float32),
            pltpu.SemaphoreType.DMA((2,)),
            pltpu.SemaphoreType.DMA((2,)),
            pltpu.SemaphoreType.DMA((2,)),
            pltpu.SemaphoreType.DMA((2,)),
            pltpu.SemaphoreType.DMA((2,)),
            pltpu.SemaphoreType.DMA((2,)),
            pltpu.SemaphoreType.DMA((2,)),
            pltpu.SemaphoreType.DMA((2,)),
        ],
        compiler_params=pltpu.CompilerParams(
            collective_id=0, vmem_limit_bytes=100 * 1024 * 1024),
    )(x, Wq, Wo, K_ext, V_ext)


# device time: 109438 ns/iter; 3.4625x vs baseline; 1.2268x over previous
import jax
import jax.numpy as jnp
from jax import lax
from jax.experimental import pallas as pl
from jax.experimental.pallas import tpu as pltpu

N_DEV = 8
B = 4
Sq = 256
Hq = 8
Hkv = 2
Dh = 128
Dm = 1024
C = 1024
G = Hq // Hkv
R = G * Sq
SCALE = 0.08838834764831843
HOPS = 3

_MESH = pl.DeviceIdType.MESH


def _gray(v):
    return jnp.where(v < 4, v, 11 - v)


def kernel(x, Wq, Wo, K_ext, V_ext):
    def body(x_ref, wq_ref, wo_ref, k_ref, v_ref, out_ref,
             rbuf, lbuf, mbuf, rl, ll, mlb, acc_run_ref,
             r_send, r_recv, l_send, l_recv,
             rl_send, rl_recv, ll_send, ll_recv,
             m_send, m_recv, ml2_send, ml2_recv):
        my = lax.axis_index("i")
        vi = _gray(my)
        right = _gray(lax.rem(vi + 1, N_DEV))
        left = _gray(lax.rem(vi + N_DEV - 1, N_DEV))
        is_even = lax.rem(vi, 2) == 0
        partner_v = lax.rem(vi + jnp.where(is_even, 3, 5), N_DEV)
        partner = _gray(partner_v)

        barrier = pltpu.get_barrier_semaphore()
        for nbr in (left, right, partner):
            pl.semaphore_signal(barrier, inc=1, device_id=(nbr,),
                                device_id_type=_MESH)
        pl.semaphore_wait(barrier, 3)

        xqT = lax.dot_general(
            wq_ref[:], x_ref[:].reshape(B * Sq, Dm),
            (((0,), (1,)), ((), ())),
            preferred_element_type=jnp.float32) * SCALE

        l_run = [[None] * Hkv for _ in range(B)]
        for b in range(B):
            for g in range(Hkv):
                qT = jnp.concatenate(
                    [xqT[(g * G + qi) * Dh:(g * G + qi + 1) * Dh,
                         b * Sq:(b + 1) * Sq]
                     for qi in range(G)], axis=1)
                kg = k_ref[b, :, g, :]
                sT = lax.dot_general(
                    kg, qT, (((1,), (0,)), ((), ())),
                    preferred_element_type=jnp.float32)
                pT = jnp.exp(sT)
                l_b = jnp.sum(pT, axis=0, keepdims=True)
                vg = v_ref[b, :, g, :]
                accT = lax.dot_general(
                    vg, pT, (((0,), (0,)), ((), ())),
                    preferred_element_type=jnp.float32)
                acc_run_ref[b, g, :, :] = accT
                acc16 = accT.astype(jnp.bfloat16)
                l16 = l_b.astype(jnp.bfloat16)
                rbuf[0, b, g, :, :] = acc16
                lbuf[0, b, g, :, :] = acc16
                rl[0, b, g, :, :] = l16
                ll[0, b, g, :, :] = l16
                l_run[b][g] = l_b

        def make(buf, sems_s, sems_r, ss, rs, dst):
            return pltpu.make_async_remote_copy(
                src_ref=buf.at[ss], dst_ref=buf.at[rs],
                send_sem=sems_s.at[ss], recv_sem=sems_r.at[rs],
                device_id=(dst,), device_id_type=_MESH)

        def make_match(src):
            return pltpu.make_async_remote_copy(
                src_ref=src, dst_ref=mbuf,
                send_sem=m_send, recv_sem=m_recv,
                device_id=(partner,), device_id_type=_MESH)

        def make_match_l(src):
            return pltpu.make_async_remote_copy(
                src_ref=src, dst_ref=mlb,
                send_sem=ml2_send, recv_sem=ml2_recv,
                device_id=(partner,), device_id_type=_MESH)

        def start_ring(h):
            ss, rs = h % 2, (h + 1) % 2
            rdmas = [make(rbuf, r_send, r_recv, ss, rs, right),
                     make(rl, rl_send, rl_recv, ss, rs, right),
                     make(lbuf, l_send, l_recv, ss, rs, left),
                     make(ll, ll_send, ll_recv, ss, rs, left)]
            for r in rdmas:
                r.start()
            return rdmas

        def merge(rs, with_match):
            for b in range(B):
                for g in range(Hkv):
                    inc = (rbuf[rs, b, g, :, :].astype(jnp.float32)
                           + lbuf[rs, b, g, :, :].astype(jnp.float32))
                    l_inc = (rl[rs, b, g, :, :].astype(jnp.float32)
                             + ll[rs, b, g, :, :].astype(jnp.float32))
                    if with_match:
                        inc = inc + mbuf[b, g, :, :].astype(jnp.float32)
                        l_inc = l_inc + mlb[b, g, :, :].astype(jnp.float32)
                    acc_run_ref[b, g, :, :] = acc_run_ref[b, g, :, :] + inc
                    l_run[b][g] = l_run[b][g] + l_inc

        inflight = start_ring(0)
        for r in inflight:
            r.wait()

        inflight = start_ring(1)

        @pl.when(is_even)
        def _():
            make_match(rbuf.at[1]).start()
            make_match_l(rl.at[1]).start()

        @pl.when(jnp.logical_not(is_even))
        def _():
            make_match(lbuf.at[1]).start()
            make_match_l(ll.at[1]).start()

        merge(1, with_match=False)
        for r in inflight:
            r.wait()
        make_match(rbuf.at[1]).wait()
        make_match_l(rl.at[1]).wait()

        inflight = start_ring(2)
        merge(0, with_match=True)
        for r in inflight:
            r.wait()
        merge(1, with_match=False)

        for b in range(B):
            total = None
            for g in range(Hkv):
                inv_l = 1.0 / l_run[b][g]
                for qi in range(G):
                    h_idx = g * G + qi
                    oT = (acc_run_ref[b, g, :, qi * Sq:(qi + 1) * Sq]
                          * inv_l[:, qi * Sq:(qi + 1) * Sq])
                    contrib = lax.dot_general(
                        oT, wo_ref[h_idx * Dh:(h_idx + 1) * Dh, :],
                        (((0,), (0,)), ((), ())),
                        preferred_element_type=jnp.float32)
                    total = contrib if total is None else total + contrib
            out_ref[b, :, :] = total

    return pl.pallas_call(
        body,
        out_shape=jax.ShapeDtypeStruct((B, Sq, Dm), jnp.float32),
        in_specs=[pl.BlockSpec(memory_space=pltpu.VMEM)] * 5,
        out_specs=pl.BlockSpec(memory_space=pltpu.VMEM),
        scratch_shapes=[
            pltpu.VMEM((2, B, Hkv, Dh, R), jnp.bfloat16),
            pltpu.VMEM((2, B, Hkv, Dh, R), jnp.bfloat16),
            pltpu.VMEM((B, Hkv, Dh, R), jnp.bfloat16),
            pltpu.VMEM((2, B, Hkv, 1, R), jnp.bfloat16),
            pltpu.VMEM((2, B, Hkv, 1, R), jnp.bfloat16),
            pltpu.VMEM((B, Hkv, 1, R), jnp.bfloat16),
            pltpu.VMEM((B, Hkv, Dh, R), jnp.float32),
            pltpu.SemaphoreType.DMA((2,)),
            pltpu.SemaphoreType.DMA((2,)),
            pltpu.SemaphoreType.DMA((2,)),
            pltpu.SemaphoreType.DMA((2,)),
            pltpu.SemaphoreType.DMA((2,)),
            pltpu.SemaphoreType.DMA((2,)),
            pltpu.SemaphoreType.DMA((2,)),
            pltpu.SemaphoreType.DMA((2,)),
            pltpu.SemaphoreType.DMA,
            pltpu.SemaphoreType.DMA,
            pltpu.SemaphoreType.DMA,
            pltpu.SemaphoreType.DMA,
        ],
        compiler_params=pltpu.CompilerParams(
            collective_id=0, vmem_limit_bytes=100 * 1024 * 1024),
    )(x, Wq, Wo, K_ext, V_ext)
